# baseline (speedup 1.0000x reference)
import functools

import jax
import jax.numpy as jnp
from jax import lax
from jax.experimental import pallas as pl
from jax.experimental.pallas import tpu as pltpu
from jax.experimental.pallas import tpu_sc as plsc

NC = 2
NS = 16
NW = NC * NS
D = 8
BN = 2048


NBUF = 4


@functools.lru_cache(maxsize=None)
def _sc_scatter(E, NP, K, with_table=True):
    EW = E // NW
    NCH = EW // K
    assert NCH % NBUF == 0 and NCH >= 2 * NBUF
    SR = NP // NS
    NV = K // 4

    mesh = plsc.VectorSubcoreMesh(core_axis_name="c", subcore_axis_name="s")

    buf_types = []
    for _ in range(NBUF):
        buf_types += [
            pltpu.VMEM((K,), jnp.int32),
            pltpu.VMEM((K,), jnp.int32),
            pltpu.VMEM((K,), jnp.float32),
            pltpu.VMEM((K, D), jnp.float32),
            pltpu.SemaphoreType.DMA,
            pltpu.SemaphoreType.DMA,
        ]

    @functools.partial(
        pl.kernel,
        mesh=mesh,
        compiler_params=pltpu.CompilerParams(
            needs_layout_passes=False, use_tc_tiling_on_sc=False),
        out_type=jax.ShapeDtypeStruct((NC, NP, D), jnp.float32),
        scratch_types=[
            pltpu.VMEM_SHARED((NP, D), jnp.float32),
            pltpu.VMEM_SHARED((NP, D), jnp.float32),
        ] + buf_types,
    )
    def sc_scatter(r_hbm, c_hbm, ew_hbm, tbl_hbm, zero_hbm, out_hbm,
                   tbl_sh, acc_sh, *bufs):
        rbuf = [bufs[6 * b + 0] for b in range(NBUF)]
        cbuf = [bufs[6 * b + 1] for b in range(NBUF)]
        ewbuf = [bufs[6 * b + 2] for b in range(NBUF)]
        rows = [bufs[6 * b + 3] for b in range(NBUF)]
        gsem = [bufs[6 * b + 4] for b in range(NBUF)]
        ssem = [bufs[6 * b + 5] for b in range(NBUF)]

        cid = lax.axis_index("c")
        sid = lax.axis_index("s")
        wid = cid * NS + sid
        e0 = wid * EW
        r0 = sid * SR
        if with_table:
            pltpu.sync_copy(tbl_hbm.at[pl.ds(r0, SR)], tbl_sh.at[pl.ds(r0, SR)])
        pltpu.sync_copy(zero_hbm.at[pl.ds(r0, SR)], acc_sh.at[pl.ds(r0, SR)])
        plsc.subcore_barrier()

        lane = lax.iota(jnp.int32, 16)
        eoff = lane >> 2
        coff = lane & 3
        zvec = jnp.zeros((16,), jnp.float32)

        def lin(j, b):
            base = e0 + j * K
            pltpu.sync_copy(r_hbm.at[pl.ds(base, K)], rbuf[b])
            pltpu.sync_copy(c_hbm.at[pl.ds(base, K)], cbuf[b])
            pltpu.sync_copy(ew_hbm.at[pl.ds(base, K)], ewbuf[b])

        def drain(b, sem):
            pltpu.make_async_copy(zero_hbm.at[pl.ds(0, K)], rows[b], sem).wait()

        if not with_table:
            def zero_rows2(v, c2):
                r16 = (lane + v * 16) >> 3
                c16 = (lane + v * 16) & 7
                for b in range(NBUF):
                    plsc.store_scatter(rows[b], [r16, c16], zvec)
                return c2
            lax.fori_loop(0, K * D // 16, zero_rows2, 0)

        lin(0, 0)
        if with_table:
            pltpu.async_copy(tbl_sh.at[rbuf[0]], rows[0], gsem[0])

        def scale_and_scatter(j, p):
            if with_table:
                drain(p, gsem[p])

                def scale(v, c2):
                    e16 = eoff + v * 4
                    ew16 = plsc.load_gather(ewbuf[p], [e16])
                    vals = plsc.load_gather(rows[p], [e16, coff])
                    plsc.store_scatter(rows[p], [e16, coff], vals * ew16)
                    return c2
            else:
                def scale(v, c2):
                    e16 = eoff + v * 4
                    ew16 = plsc.load_gather(ewbuf[p], [e16])
                    plsc.store_scatter(rows[p], [e16, coff], ew16)
                    return c2

            lax.fori_loop(0, NV, scale, 0, unroll=4)
            pltpu.async_copy(rows[p], acc_sh.at[cbuf[p]], ssem[p], add=True)

        def quad(i4, carry):
            for b in range(NBUF):
                j = i4 * NBUF + b
                q = (b + 1) % NBUF

                if b == NBUF - 1:
                    drain(q, ssem[q])
                else:
                    @pl.when(i4 > 0)
                    def _():
                        drain(q, ssem[q])

                if b == NBUF - 1:
                    @pl.when(i4 < NCH // NBUF - 1)
                    def _():
                        lin(j + 1, q)
                        if with_table:
                            pltpu.async_copy(tbl_sh.at[rbuf[q]], rows[q],
                                             gsem[q])
                else:
                    lin(j + 1, q)
                    if with_table:
                        pltpu.async_copy(tbl_sh.at[rbuf[q]], rows[q], gsem[q])

                scale_and_scatter(j, b)
            return carry

        lax.fori_loop(0, NCH // NBUF, quad, 0)
        for b in range(1, NBUF):
            drain((NCH - NBUF + b) % NBUF, ssem[(NCH - NBUF + b) % NBUF])

        plsc.subcore_barrier()
        pltpu.sync_copy(acc_sh.at[pl.ds(r0, SR)],
                        out_hbm.at[cid, pl.ds(r0, SR)])

    return sc_scatter


def _tc1_body(deg2_ref, x8_ref, dinv_ref, tp1_ref):
    deg = deg2_ref[0, :, 0:1] + deg2_ref[1, :, 0:1] + 1.0
    dinv = jnp.where(deg > 0, lax.rsqrt(jnp.maximum(deg, 1e-12)), 0.0)
    dinv_ref[...] = dinv
    tp1_ref[...] = x8_ref[...] * dinv


def _tc2_body(p1_ref, dinv_ref, x8_ref, w1_ref, b1_ref, w2_ref, tp2_ref):
    dinv = dinv_ref[...]
    agg = (p1_ref[0] + p1_ref[1]) * dinv + dinv * dinv * x8_ref[...]
    h = jnp.zeros((agg.shape[0], 16), jnp.float32) + b1_ref[...]
    for k in range(4):
        h = h + agg[:, k:k + 1] * w1_ref[k:k + 1, :]
    h = jnp.maximum(h, 0.0)
    t2 = jnp.zeros((agg.shape[0], D), jnp.float32)
    for k in range(16):
        t2 = t2 + h[:, k:k + 1] * w2_ref[k:k + 1, :]
    tp2_ref[...] = t2 * dinv


def _tc3_body(p2_ref, dinv_ref, tp2_ref, b2_ref, out_ref):
    dinv = dinv_ref[...]
    out_ref[...] = ((p2_ref[0, :, 0:4] + p2_ref[1, :, 0:4]) * dinv
                    + tp2_ref[:, 0:4] * dinv + b2_ref[...])


def _full(shape):
    nd = len(shape)
    return pl.BlockSpec(shape, lambda i: (0,) * nd)


def _tc1(deg2, x8):
    NP = x8.shape[0]
    return pl.pallas_call(
        _tc1_body,
        grid=(NP // BN,),
        in_specs=[pl.BlockSpec((2, BN, D), lambda i: (0, i, 0)),
                  pl.BlockSpec((BN, D), lambda i: (i, 0))],
        out_specs=[pl.BlockSpec((BN, 1), lambda i: (i, 0)),
                   pl.BlockSpec((BN, D), lambda i: (i, 0))],
        out_shape=[jax.ShapeDtypeStruct((NP, 1), jnp.float32),
                   jax.ShapeDtypeStruct((NP, D), jnp.float32)],
    )(deg2, x8)


def _tc2(p1, dinv, x8, W1p, b1, W2p):
    NP = x8.shape[0]
    return pl.pallas_call(
        _tc2_body,
        grid=(NP // BN,),
        in_specs=[pl.BlockSpec((2, BN, D), lambda i: (0, i, 0)),
                  pl.BlockSpec((BN, 1), lambda i: (i, 0)),
                  pl.BlockSpec((BN, D), lambda i: (i, 0)),
                  _full((D, 16)), _full((1, 16)), _full((16, D))],
        out_specs=pl.BlockSpec((BN, D), lambda i: (i, 0)),
        out_shape=jax.ShapeDtypeStruct((NP, D), jnp.float32),
    )(p1, dinv, x8, W1p, b1, W2p)


def _tc3(p2, dinv, tp2, b2):
    NP = tp2.shape[0]
    return pl.pallas_call(
        _tc3_body,
        grid=(NP // BN,),
        in_specs=[pl.BlockSpec((2, BN, D), lambda i: (0, i, 0)),
                  pl.BlockSpec((BN, 1), lambda i: (i, 0)),
                  pl.BlockSpec((BN, D), lambda i: (i, 0)),
                  _full((1, 4))],
        out_specs=pl.BlockSpec((BN, 4), lambda i: (i, 0)),
        out_shape=jax.ShapeDtypeStruct((NP, 4), jnp.float32),
    )(p2, dinv, tp2, b2)


def kernel(x, edge_index, edge_attr, W1, b1, W2, b2):
    N = x.shape[0]
    E = edge_attr.shape[0]
    NP = 100352
    K = 200

    ei = edge_index.astype(jnp.int32)
    r, c = ei[0], ei[1]
    ew = edge_attr.astype(jnp.float32)
    x8 = jnp.pad(x, ((0, NP - N), (0, D - 3)))
    ones8 = jnp.pad(jnp.ones((NP, 4), jnp.float32), ((0, 0), (0, D - 4)))
    zeros8 = jnp.zeros((NP, D), jnp.float32)
    W1p = jnp.pad(W1, ((0, D - 3), (0, 0)))
    W2p = jnp.pad(W2, ((0, 0), (0, D - 4)))

    sc = _sc_scatter(E, NP, K)
    sc_deg = _sc_scatter(E, NP, K, with_table=False)
    deg2 = sc_deg(r, c, ew, ones8, zeros8)
    dinv, tp1 = _tc1(deg2, x8)
    p1 = sc(r, c, ew, tp1, zeros8)
    tp2 = _tc2(p1, dinv, x8, W1p, b1.reshape(1, 16), W2p)
    p2 = sc(r, c, ew, tp2, zeros8)
    out = _tc3(p2, dinv, tp2, b2.reshape(1, 4))
    return out[:N]


# speedup vs baseline: 1.7558x; 1.7558x over previous
import functools

import jax
import jax.numpy as jnp
from jax import lax
from jax.experimental import pallas as pl
from jax.experimental.pallas import tpu as pltpu
from jax.experimental.pallas import tpu_sc as plsc

NC = 2
NS = 16
NW = NC * NS
D = 8
BN = 2048


NBUF = 2


@functools.lru_cache(maxsize=None)
def _sc_scatter(E, NP, K, with_table=True):
    EW = E // NW
    NCH = EW // K
    assert NCH % NBUF == 0 and NCH >= 2 * NBUF
    SR = NP // NS
    NV = K // 4

    mesh = plsc.VectorSubcoreMesh(core_axis_name="c", subcore_axis_name="s")

    buf_types = []
    for _ in range(NBUF):
        buf_types += [
            pltpu.VMEM((K,), jnp.int32),
            pltpu.VMEM((K,), jnp.int32),
            pltpu.VMEM((K,), jnp.float32),
            pltpu.VMEM((K, D), jnp.float32),
            pltpu.SemaphoreType.DMA,
            pltpu.SemaphoreType.DMA,
        ]

    @functools.partial(
        pl.kernel,
        mesh=mesh,
        compiler_params=pltpu.CompilerParams(
            needs_layout_passes=False, use_tc_tiling_on_sc=False),
        out_type=jax.ShapeDtypeStruct((NC, NP, D), jnp.float32),
        scratch_types=[
            pltpu.VMEM_SHARED((NP, D), jnp.float32),
            pltpu.VMEM_SHARED((NP, D), jnp.float32),
        ] + buf_types,
    )
    def sc_scatter(r_hbm, c_hbm, ew_hbm, tbl_hbm, zero_hbm, out_hbm,
                   tbl_sh, acc_sh, *bufs):
        rbuf = [bufs[6 * b + 0] for b in range(NBUF)]
        cbuf = [bufs[6 * b + 1] for b in range(NBUF)]
        ewbuf = [bufs[6 * b + 2] for b in range(NBUF)]
        rows = [bufs[6 * b + 3] for b in range(NBUF)]
        gsem = [bufs[6 * b + 4] for b in range(NBUF)]
        ssem = [bufs[6 * b + 5] for b in range(NBUF)]

        cid = lax.axis_index("c")
        sid = lax.axis_index("s")
        wid = cid * NS + sid
        e0 = wid * EW
        r0 = sid * SR
        if with_table:
            pltpu.sync_copy(tbl_hbm.at[pl.ds(r0, SR)], tbl_sh.at[pl.ds(r0, SR)])
        pltpu.sync_copy(zero_hbm.at[pl.ds(r0, SR)], acc_sh.at[pl.ds(r0, SR)])
        plsc.subcore_barrier()

        lane = lax.iota(jnp.int32, 16)
        eoff = lane >> 2
        coff = lane & 3
        zvec = jnp.zeros((16,), jnp.float32)

        def lin(j, b):
            base = e0 + j * K
            pltpu.sync_copy(r_hbm.at[pl.ds(base, K)], rbuf[b])
            pltpu.sync_copy(c_hbm.at[pl.ds(base, K)], cbuf[b])
            pltpu.sync_copy(ew_hbm.at[pl.ds(base, K)], ewbuf[b])

        def drain(b, sem):
            pltpu.make_async_copy(zero_hbm.at[pl.ds(0, K)], rows[b], sem).wait()

        if not with_table:
            def zero_rows2(v, c2):
                r16 = (lane + v * 16) >> 3
                c16 = (lane + v * 16) & 7
                for b in range(NBUF):
                    plsc.store_scatter(rows[b], [r16, c16], zvec)
                return c2
            lax.fori_loop(0, K * D // 16, zero_rows2, 0)

        lin(0, 0)
        if with_table:
            pltpu.async_copy(tbl_sh.at[rbuf[0]], rows[0], gsem[0])

        def scale_and_scatter(j, p):
            if with_table:
                drain(p, gsem[p])

                def scale(v, c2):
                    e16 = eoff + v * 4
                    ew16 = plsc.load_gather(ewbuf[p], [e16])
                    vals = plsc.load_gather(rows[p], [e16, coff])
                    plsc.store_scatter(rows[p], [e16, coff], vals * ew16)
                    return c2
            else:
                def scale(v, c2):
                    e16 = eoff + v * 4
                    ew16 = plsc.load_gather(ewbuf[p], [e16])
                    plsc.store_scatter(rows[p], [e16, coff], ew16)
                    return c2

            lax.fori_loop(0, NV, scale, 0, unroll=4)
            pltpu.async_copy(rows[p], acc_sh.at[cbuf[p]], ssem[p], add=True)

        def quad(i4, carry):
            for b in range(NBUF):
                j = i4 * NBUF + b
                q = (b + 1) % NBUF

                if b == NBUF - 1:
                    drain(q, ssem[q])
                else:
                    @pl.when(i4 > 0)
                    def _():
                        drain(q, ssem[q])

                if b == NBUF - 1:
                    @pl.when(i4 < NCH // NBUF - 1)
                    def _():
                        lin(j + 1, q)
                        if with_table:
                            pltpu.async_copy(tbl_sh.at[rbuf[q]], rows[q],
                                             gsem[q])
                else:
                    lin(j + 1, q)
                    if with_table:
                        pltpu.async_copy(tbl_sh.at[rbuf[q]], rows[q], gsem[q])

                scale_and_scatter(j, b)
            return carry

        lax.fori_loop(0, NCH // NBUF, quad, 0)
        for b in range(1, NBUF):
            drain((NCH - NBUF + b) % NBUF, ssem[(NCH - NBUF + b) % NBUF])

        plsc.subcore_barrier()
        pltpu.sync_copy(acc_sh.at[pl.ds(r0, SR)],
                        out_hbm.at[cid, pl.ds(r0, SR)])

    return sc_scatter


def _tc1_body(deg2_ref, x8_ref, dinv_ref, tp1_ref):
    deg = deg2_ref[0, :, 0:1] + deg2_ref[1, :, 0:1] + 1.0
    dinv = jnp.where(deg > 0, lax.rsqrt(jnp.maximum(deg, 1e-12)), 0.0)
    dinv_ref[...] = dinv
    tp1_ref[...] = x8_ref[...] * dinv


def _tc2_body(p1_ref, dinv_ref, x8_ref, w1_ref, b1_ref, w2_ref, tp2_ref):
    dinv = dinv_ref[...]
    agg = (p1_ref[0] + p1_ref[1]) * dinv + dinv * dinv * x8_ref[...]
    h = jnp.zeros((agg.shape[0], 16), jnp.float32) + b1_ref[...]
    for k in range(4):
        h = h + agg[:, k:k + 1] * w1_ref[k:k + 1, :]
    h = jnp.maximum(h, 0.0)
    t2 = jnp.zeros((agg.shape[0], D), jnp.float32)
    for k in range(16):
        t2 = t2 + h[:, k:k + 1] * w2_ref[k:k + 1, :]
    tp2_ref[...] = t2 * dinv


def _tc3_body(p2_ref, dinv_ref, tp2_ref, b2_ref, out_ref):
    dinv = dinv_ref[...]
    out_ref[...] = ((p2_ref[0, :, 0:4] + p2_ref[1, :, 0:4]) * dinv
                    + tp2_ref[:, 0:4] * dinv + b2_ref[...])


def _full(shape):
    nd = len(shape)
    return pl.BlockSpec(shape, lambda i: (0,) * nd)


def _tc1(deg2, x8):
    NP = x8.shape[0]
    return pl.pallas_call(
        _tc1_body,
        grid=(NP // BN,),
        in_specs=[pl.BlockSpec((2, BN, D), lambda i: (0, i, 0)),
                  pl.BlockSpec((BN, D), lambda i: (i, 0))],
        out_specs=[pl.BlockSpec((BN, 1), lambda i: (i, 0)),
                   pl.BlockSpec((BN, D), lambda i: (i, 0))],
        out_shape=[jax.ShapeDtypeStruct((NP, 1), jnp.float32),
                   jax.ShapeDtypeStruct((NP, D), jnp.float32)],
    )(deg2, x8)


def _tc2(p1, dinv, x8, W1p, b1, W2p):
    NP = x8.shape[0]
    return pl.pallas_call(
        _tc2_body,
        grid=(NP // BN,),
        in_specs=[pl.BlockSpec((2, BN, D), lambda i: (0, i, 0)),
                  pl.BlockSpec((BN, 1), lambda i: (i, 0)),
                  pl.BlockSpec((BN, D), lambda i: (i, 0)),
                  _full((D, 16)), _full((1, 16)), _full((16, D))],
        out_specs=pl.BlockSpec((BN, D), lambda i: (i, 0)),
        out_shape=jax.ShapeDtypeStruct((NP, D), jnp.float32),
    )(p1, dinv, x8, W1p, b1, W2p)


def _tc3(p2, dinv, tp2, b2):
    NP = tp2.shape[0]
    return pl.pallas_call(
        _tc3_body,
        grid=(NP // BN,),
        in_specs=[pl.BlockSpec((2, BN, D), lambda i: (0, i, 0)),
                  pl.BlockSpec((BN, 1), lambda i: (i, 0)),
                  pl.BlockSpec((BN, D), lambda i: (i, 0)),
                  _full((1, 4))],
        out_specs=pl.BlockSpec((BN, 4), lambda i: (i, 0)),
        out_shape=jax.ShapeDtypeStruct((NP, 4), jnp.float32),
    )(p2, dinv, tp2, b2)


def kernel(x, edge_index, edge_attr, W1, b1, W2, b2):
    N = x.shape[0]
    E = edge_attr.shape[0]
    NP = 100352
    K = 1000

    ei = edge_index.astype(jnp.int32)
    r, c = ei[0], ei[1]
    ew = edge_attr.astype(jnp.float32)
    x8 = jnp.pad(x, ((0, NP - N), (0, D - 3)))
    ones8 = jnp.pad(jnp.ones((NP, 4), jnp.float32), ((0, 0), (0, D - 4)))
    zeros8 = jnp.zeros((NP, D), jnp.float32)
    W1p = jnp.pad(W1, ((0, D - 3), (0, 0)))
    W2p = jnp.pad(W2, ((0, 0), (0, D - 4)))

    sc = _sc_scatter(E, NP, K)
    sc_deg = _sc_scatter(E, NP, K, with_table=False)
    deg2 = sc_deg(r, c, ew, ones8, zeros8)
    dinv, tp1 = _tc1(deg2, x8)
    p1 = sc(r, c, ew, tp1, zeros8)
    tp2 = _tc2(p1, dinv, x8, W1p, b1.reshape(1, 16), W2p)
    p2 = sc(r, c, ew, tp2, zeros8)
    out = _tc3(p2, dinv, tp2, b2.reshape(1, 4))
    return out[:N]


# speedup vs baseline: 2.1020x; 1.1972x over previous
import functools

import jax
import jax.numpy as jnp
from jax import lax
from jax.experimental import pallas as pl
from jax.experimental.pallas import tpu as pltpu
from jax.experimental.pallas import tpu_sc as plsc

NC = 2
NS = 16
NW = NC * NS
D = 8
BN = 2048


NBUF = 2


@functools.lru_cache(maxsize=None)
def _sc_scatter(E, NP, K):
    EW = E // NW
    NCH = EW // K
    assert NCH % NBUF == 0 and NCH >= 2 * NBUF
    SR = NP // NS
    NV = K // 4

    mesh = plsc.VectorSubcoreMesh(core_axis_name="c", subcore_axis_name="s")

    buf_types = []
    for _ in range(NBUF):
        buf_types += [
            pltpu.VMEM((K,), jnp.int32),
            pltpu.VMEM((K,), jnp.int32),
            pltpu.VMEM((K,), jnp.float32),
            pltpu.VMEM((K, D), jnp.float32),
            pltpu.SemaphoreType.DMA,
            pltpu.SemaphoreType.DMA,
        ]

    @functools.partial(
        pl.kernel,
        mesh=mesh,
        compiler_params=pltpu.CompilerParams(
            needs_layout_passes=False, use_tc_tiling_on_sc=False),
        out_type=jax.ShapeDtypeStruct((NC, NP, D), jnp.float32),
        scratch_types=[
            pltpu.VMEM_SHARED((NP, D), jnp.float32),
            pltpu.VMEM_SHARED((NP, D), jnp.float32),
        ] + buf_types,
    )
    def sc_scatter(r_hbm, c_hbm, ew_hbm, tbl_hbm, zero_hbm, out_hbm,
                   tbl_sh, acc_sh, *bufs):
        rbuf = [bufs[6 * b + 0] for b in range(NBUF)]
        cbuf = [bufs[6 * b + 1] for b in range(NBUF)]
        ewbuf = [bufs[6 * b + 2] for b in range(NBUF)]
        rows = [bufs[6 * b + 3] for b in range(NBUF)]
        gsem = [bufs[6 * b + 4] for b in range(NBUF)]
        ssem = [bufs[6 * b + 5] for b in range(NBUF)]

        cid = lax.axis_index("c")
        sid = lax.axis_index("s")
        wid = cid * NS + sid
        e0 = wid * EW
        r0 = sid * SR
        pltpu.sync_copy(tbl_hbm.at[pl.ds(r0, SR)], tbl_sh.at[pl.ds(r0, SR)])
        pltpu.sync_copy(zero_hbm.at[pl.ds(r0, SR)], acc_sh.at[pl.ds(r0, SR)])
        plsc.subcore_barrier()

        lane = lax.iota(jnp.int32, 16)
        eoff = lane >> 2
        coff = lane & 3
        def lin(j, b):
            base = e0 + j * K
            pltpu.sync_copy(r_hbm.at[pl.ds(base, K)], rbuf[b])
            pltpu.sync_copy(c_hbm.at[pl.ds(base, K)], cbuf[b])
            pltpu.sync_copy(ew_hbm.at[pl.ds(base, K)], ewbuf[b])

        def drain(b, sem):
            pltpu.make_async_copy(zero_hbm.at[pl.ds(0, K)], rows[b], sem).wait()

        lin(0, 0)
        pltpu.async_copy(tbl_sh.at[rbuf[0]], rows[0], gsem[0])

        def scale_and_scatter(j, p):
            drain(p, gsem[p])

            def scale(v, c2):
                e16 = eoff + v * 4
                ew16 = plsc.load_gather(ewbuf[p], [e16])
                vals = plsc.load_gather(rows[p], [e16, coff])
                plsc.store_scatter(rows[p], [e16, coff], vals * ew16)
                return c2

            lax.fori_loop(0, NV, scale, 0, unroll=4)
            pltpu.async_copy(rows[p], acc_sh.at[cbuf[p]], ssem[p], add=True)

        def quad(i4, carry):
            for b in range(NBUF):
                j = i4 * NBUF + b
                q = (b + 1) % NBUF

                if b == NBUF - 1:
                    drain(q, ssem[q])
                else:
                    @pl.when(i4 > 0)
                    def _():
                        drain(q, ssem[q])

                if b == NBUF - 1:
                    @pl.when(i4 < NCH // NBUF - 1)
                    def _():
                        lin(j + 1, q)
                        pltpu.async_copy(tbl_sh.at[rbuf[q]], rows[q], gsem[q])
                else:
                    lin(j + 1, q)
                    pltpu.async_copy(tbl_sh.at[rbuf[q]], rows[q], gsem[q])

                scale_and_scatter(j, b)
            return carry

        lax.fori_loop(0, NCH // NBUF, quad, 0)
        for b in range(1, NBUF):
            drain((NCH - NBUF + b) % NBUF, ssem[(NCH - NBUF + b) % NBUF])

        plsc.subcore_barrier()
        pltpu.sync_copy(acc_sh.at[pl.ds(r0, SR)],
                        out_hbm.at[cid, pl.ds(r0, SR)])

    return sc_scatter


@functools.lru_cache(maxsize=None)
def _sc_degree(E, NP, K):
    EW = E // NW
    NCH = EW // K
    assert NCH % 2 == 0 and K % 16 == 0
    NV16 = K // 16

    mesh = plsc.VectorSubcoreMesh(core_axis_name="c", subcore_axis_name="s")

    @functools.partial(
        pl.kernel,
        mesh=mesh,
        compiler_params=pltpu.CompilerParams(
            needs_layout_passes=False, use_tc_tiling_on_sc=False),
        out_type=jax.ShapeDtypeStruct((NW, NP), jnp.float32),
        scratch_types=[
            pltpu.VMEM((NP,), jnp.float32),
            pltpu.VMEM((K,), jnp.int32),
            pltpu.VMEM((K,), jnp.float32),
            pltpu.VMEM((K,), jnp.int32),
            pltpu.VMEM((K,), jnp.float32),
        ],
    )
    def sc_degree(c_hbm, ew_hbm, out_hbm, deg, cb0, eb0, cb1, eb1):
        cid = lax.axis_index("c")
        sid = lax.axis_index("s")
        wid = cid * NS + sid
        e0 = wid * EW
        cbuf = [cb0, cb1]
        ebuf = [eb0, eb1]

        zvec = jnp.zeros((16,), jnp.float32)

        def zero(v, carry):
            deg[pl.ds(v * 16, 16)] = zvec
            return carry

        lax.fori_loop(0, NP // 16, zero, 0, unroll=8)

        def lin(j, b):
            base = e0 + j * K
            pltpu.sync_copy(c_hbm.at[pl.ds(base, K)], cbuf[b])
            pltpu.sync_copy(ew_hbm.at[pl.ds(base, K)], ebuf[b])

        lin(0, 0)

        def pair(i2, carry):
            for b in range(2):
                j = 2 * i2 + b

                @pl.when(j < NCH - 1)
                def _():
                    lin(j + 1, 1 - b)

                def accum(v, c2):
                    c16 = cbuf[b][pl.ds(v * 16, 16)]
                    ew16 = ebuf[b][pl.ds(v * 16, 16)]
                    plsc.addupdate_scatter(deg, [c16], ew16)
                    return c2

                lax.fori_loop(0, NV16, accum, 0, unroll=4)
            return carry

        lax.fori_loop(0, NCH // 2, pair, 0)
        pltpu.sync_copy(deg, out_hbm.at[wid])

    return sc_degree


def _tc1_body(degp_ref, x8_ref, dinv_ref, tp1_ref):
    ones = jnp.ones((NW, 1), jnp.float32)
    deg = lax.dot_general(degp_ref[...], ones, (((0,), (0,)), ((), ())),
                          preferred_element_type=jnp.float32) + 1.0
    dinv = jnp.where(deg > 0, lax.rsqrt(jnp.maximum(deg, 1e-12)), 0.0)
    dinv_ref[...] = dinv
    tp1_ref[...] = x8_ref[...] * dinv


def _tc2_body(p1_ref, dinv_ref, x8_ref, w1_ref, b1_ref, w2_ref, tp2_ref):
    dinv = dinv_ref[...]
    agg = (p1_ref[0] + p1_ref[1]) * dinv + dinv * dinv * x8_ref[...]
    h = jnp.zeros((agg.shape[0], 16), jnp.float32) + b1_ref[...]
    for k in range(4):
        h = h + agg[:, k:k + 1] * w1_ref[k:k + 1, :]
    h = jnp.maximum(h, 0.0)
    t2 = jnp.zeros((agg.shape[0], D), jnp.float32)
    for k in range(16):
        t2 = t2 + h[:, k:k + 1] * w2_ref[k:k + 1, :]
    tp2_ref[...] = t2 * dinv


def _tc3_body(p2_ref, dinv_ref, tp2_ref, b2_ref, out_ref):
    dinv = dinv_ref[...]
    out_ref[...] = ((p2_ref[0, :, 0:4] + p2_ref[1, :, 0:4]) * dinv
                    + tp2_ref[:, 0:4] * dinv + b2_ref[...])


def _full(shape):
    nd = len(shape)
    return pl.BlockSpec(shape, lambda i: (0,) * nd)


def _tc1(degp, x8):
    NP = x8.shape[0]
    return pl.pallas_call(
        _tc1_body,
        grid=(NP // BN,),
        in_specs=[pl.BlockSpec((NW, BN), lambda i: (0, i)),
                  pl.BlockSpec((BN, D), lambda i: (i, 0))],
        out_specs=[pl.BlockSpec((BN, 1), lambda i: (i, 0)),
                   pl.BlockSpec((BN, D), lambda i: (i, 0))],
        out_shape=[jax.ShapeDtypeStruct((NP, 1), jnp.float32),
                   jax.ShapeDtypeStruct((NP, D), jnp.float32)],
    )(degp, x8)


def _tc2(p1, dinv, x8, W1p, b1, W2p):
    NP = x8.shape[0]
    return pl.pallas_call(
        _tc2_body,
        grid=(NP // BN,),
        in_specs=[pl.BlockSpec((2, BN, D), lambda i: (0, i, 0)),
                  pl.BlockSpec((BN, 1), lambda i: (i, 0)),
                  pl.BlockSpec((BN, D), lambda i: (i, 0)),
                  _full((D, 16)), _full((1, 16)), _full((16, D))],
        out_specs=pl.BlockSpec((BN, D), lambda i: (i, 0)),
        out_shape=jax.ShapeDtypeStruct((NP, D), jnp.float32),
    )(p1, dinv, x8, W1p, b1, W2p)


def _tc3(p2, dinv, tp2, b2):
    NP = tp2.shape[0]
    return pl.pallas_call(
        _tc3_body,
        grid=(NP // BN,),
        in_specs=[pl.BlockSpec((2, BN, D), lambda i: (0, i, 0)),
                  pl.BlockSpec((BN, 1), lambda i: (i, 0)),
                  pl.BlockSpec((BN, D), lambda i: (i, 0)),
                  _full((1, 4))],
        out_specs=pl.BlockSpec((BN, 4), lambda i: (i, 0)),
        out_shape=jax.ShapeDtypeStruct((NP, 4), jnp.float32),
    )(p2, dinv, tp2, b2)


def kernel(x, edge_index, edge_attr, W1, b1, W2, b2):
    N = x.shape[0]
    E = edge_attr.shape[0]
    NP = 100352
    K = 1000

    ei = edge_index.astype(jnp.int32)
    r, c = ei[0], ei[1]
    ew = edge_attr.astype(jnp.float32)
    x8 = jnp.pad(x, ((0, NP - N), (0, D - 3)))
    zeros8 = jnp.zeros((NP, D), jnp.float32)
    W1p = jnp.pad(W1, ((0, D - 3), (0, 0)))
    W2p = jnp.pad(W2, ((0, 0), (0, D - 4)))

    sc = _sc_scatter(E, NP, K)
    degp = _sc_degree(E, NP, 2000)(c, ew)
    dinv, tp1 = _tc1(degp, x8)
    p1 = sc(r, c, ew, tp1, zeros8)
    tp2 = _tc2(p1, dinv, x8, W1p, b1.reshape(1, 16), W2p)
    p2 = sc(r, c, ew, tp2, zeros8)
    out = _tc3(p2, dinv, tp2, b2.reshape(1, 4))
    return out[:N]
